# baseline (speedup 1.0000x reference)
import functools

import jax
import jax.numpy as jnp
from jax import lax
from jax.experimental import pallas as pl
from jax.experimental.pallas import tpu as pltpu
from jax.experimental.pallas import tpu_sc as plsc

NUM_WORKERS = 32
LANES = 16

CH = 16
NBUF = 2


@functools.partial(jax.jit, static_argnames=("num_codebooks", "vocab_size"))
def _sc_embedding_gather(flat_ids, table, *, num_codebooks, vocab_size):
    total_rows, = flat_ids.shape
    _, hidden = table.shape
    rows_per_w = total_rows // NUM_WORKERS
    nchunks = rows_per_w // CH
    assert rows_per_w * NUM_WORKERS == total_rows
    assert nchunks * CH == rows_per_w
    assert nchunks % NBUF == 0 and nchunks // NBUF >= 2

    mesh = plsc.VectorSubcoreMesh(core_axis_name="c", subcore_axis_name="s")

    @functools.partial(
        pl.kernel,
        mesh=mesh,
        out_type=jax.ShapeDtypeStruct((total_rows, hidden), jnp.float32),
        scratch_types=(
            [pltpu.VMEM((rows_per_w,), jnp.int32)]
            + [pltpu.VMEM((CH, hidden), jnp.float32) for _ in range(NBUF)]
            + [pltpu.SemaphoreType.DMA for _ in range(2 * NBUF)]
        ),
    )
    def body(ids_hbm, table_hbm, out_hbm, idx_v, *bufs_and_sems):
        rows = bufs_and_sems[:NBUF]
        gsem = bufs_and_sems[NBUF:2 * NBUF]
        ssem = bufs_and_sems[2 * NBUF:]
        wid = lax.axis_index("s") * 2 + lax.axis_index("c")
        base = wid * rows_per_w

        pltpu.sync_copy(ids_hbm.at[pl.ds(base, rows_per_w)], idx_v)

        lane = lax.iota(jnp.int32, LANES)

        def add_offsets(i, _):
            pos = base + i * LANES + lane
            cb = lax.rem(pos, num_codebooks)
            sl = pl.ds(i * LANES, LANES)
            idx_v[sl] = idx_v[sl] + cb * vocab_size
            return 0

        lax.fori_loop(0, rows_per_w // LANES, add_offsets, 0, unroll=4)

        def gather_start(b, chunk):
            pltpu.make_async_copy(
                table_hbm.at[idx_v.at[pl.ds(chunk * CH, CH)]], rows[b], gsem[b]
            ).start()

        def gather_wait(b):
            pltpu.make_async_copy(
                table_hbm.at[pl.ds(0, CH)], rows[b], gsem[b]
            ).wait()

        def store_start(b, chunk):
            pltpu.make_async_copy(
                rows[b], out_hbm.at[pl.ds(base + chunk * CH, CH)], ssem[b]
            ).start()

        def store_wait(b):
            pltpu.make_async_copy(
                rows[b], out_hbm.at[pl.ds(0, CH)], ssem[b]
            ).wait()

        for b in range(NBUF):
            gather_start(b, b)

        def group(gbase, _):
            for b in range(NBUF):
                chunk = gbase + b
                gather_wait(b)
                store_start(b, chunk)
                store_wait(b)
                gather_start(b, chunk + NBUF)
            return 0

        lax.fori_loop(0, (nchunks - NBUF) // NBUF,
                      lambda i, c: group(i * NBUF, c), 0)

        for b in range(NBUF):
            chunk = nchunks - NBUF + b
            gather_wait(b)
            store_start(b, chunk)
        for b in range(NBUF):
            store_wait(b)

    return body(flat_ids, table)


def kernel(input_ids, embed_weight):
    batch, num_codebooks = input_ids.shape
    table_rows, hidden = embed_weight.shape
    vocab_size = table_rows // num_codebooks
    flat_ids = input_ids.reshape(-1).astype(jnp.int32)
    out = _sc_embedding_gather(
        flat_ids, embed_weight,
        num_codebooks=num_codebooks, vocab_size=vocab_size,
    )
    return out.reshape(batch, num_codebooks, hidden)


# speedup vs baseline: 1.0028x; 1.0028x over previous
import functools

import jax
import jax.numpy as jnp
from jax import lax
from jax.experimental import pallas as pl
from jax.experimental.pallas import tpu as pltpu
from jax.experimental.pallas import tpu_sc as plsc

NUM_WORKERS = 32
LANES = 16

CH = 8
NBUF = 4
OFF = NBUF // 2


@functools.partial(jax.jit, static_argnames=("num_codebooks", "vocab_size"))
def _sc_embedding_gather(flat_ids, table, *, num_codebooks, vocab_size):
    total_rows, = flat_ids.shape
    _, hidden = table.shape
    rows_per_w = total_rows // NUM_WORKERS
    nchunks = rows_per_w // CH
    assert rows_per_w * NUM_WORKERS == total_rows
    assert nchunks * CH == rows_per_w
    assert nchunks % NBUF == 0 and nchunks // NBUF >= 2

    mesh = plsc.VectorSubcoreMesh(core_axis_name="c", subcore_axis_name="s")

    @functools.partial(
        pl.kernel,
        mesh=mesh,
        out_type=jax.ShapeDtypeStruct((total_rows, hidden), jnp.float32),
        scratch_types=(
            [pltpu.VMEM((rows_per_w,), jnp.int32)]
            + [pltpu.VMEM((CH, hidden), jnp.float32) for _ in range(NBUF)]
            + [pltpu.SemaphoreType.DMA for _ in range(2 * NBUF)]
        ),
    )
    def body(ids_hbm, table_hbm, out_hbm, idx_v, *bufs_and_sems):
        rows = bufs_and_sems[:NBUF]
        gsem = bufs_and_sems[NBUF:2 * NBUF]
        ssem = bufs_and_sems[2 * NBUF:]
        wid = lax.axis_index("s") * 2 + lax.axis_index("c")
        base = wid * rows_per_w

        pltpu.sync_copy(ids_hbm.at[pl.ds(base, rows_per_w)], idx_v)

        lane = lax.iota(jnp.int32, LANES)

        def add_offsets(i, _):
            pos = base + i * LANES + lane
            cb = lax.rem(pos, num_codebooks)
            sl = pl.ds(i * LANES, LANES)
            idx_v[sl] = idx_v[sl] + cb * vocab_size
            return 0

        lax.fori_loop(0, rows_per_w // LANES, add_offsets, 0, unroll=4)

        def gather_start(b, chunk):
            pltpu.make_async_copy(
                table_hbm.at[idx_v.at[pl.ds(chunk * CH, CH)]], rows[b], gsem[b]
            ).start()

        def gather_wait(b):
            pltpu.make_async_copy(
                table_hbm.at[pl.ds(0, CH)], rows[b], gsem[b]
            ).wait()

        def store_start(b, chunk):
            pltpu.make_async_copy(
                rows[b], out_hbm.at[pl.ds(base + chunk * CH, CH)], ssem[b]
            ).start()

        def store_wait(b):
            pltpu.make_async_copy(
                rows[b], out_hbm.at[pl.ds(0, CH)], ssem[b]
            ).wait()

        ngroups = nchunks // NBUF

        def step(n_static, gbase, b, first, last):
            gather_wait(b)
            store_start(b, gbase + b)
            bo = (b + OFF) % NBUF
            if not (first and b < OFF):
                store_wait(bo)
            if not (last and b >= OFF):
                gather_start(bo, gbase + b + OFF)

        for b in range(OFF):
            gather_start(b, b)

        for b in range(NBUF):
            step(b, 0, b, first=True, last=False)

        def group(gbase, _):
            for b in range(NBUF):
                step(None, gbase, b, first=False, last=False)
            return 0

        lax.fori_loop(1, ngroups - 1,
                      lambda g, c: group(g * NBUF, c), 0)

        for b in range(NBUF):
            step(b, nchunks - NBUF, b, first=False, last=True)
        for b in range(OFF):
            store_wait((NBUF - OFF + b) % NBUF)

    return body(flat_ids, table)


def kernel(input_ids, embed_weight):
    batch, num_codebooks = input_ids.shape
    table_rows, hidden = embed_weight.shape
    vocab_size = table_rows // num_codebooks
    flat_ids = input_ids.reshape(-1).astype(jnp.int32)
    out = _sc_embedding_gather(
        flat_ids, embed_weight,
        num_codebooks=num_codebooks, vocab_size=vocab_size,
    )
    return out.reshape(batch, num_codebooks, hidden)
